# baseline (speedup 1.0000x reference)
import jax
import jax.numpy as jnp
from jax import lax
from jax.experimental import pallas as pl
from jax.experimental.pallas import tpu as pltpu
from jax.experimental.pallas import tpu_sc as plsc

N_USER = 10000
N_ITEM = 10000
D = 128
E = 160000

L = 16
NC = 2
NS = 16
B = 80
ROWS = E // B
ROWS_PER_SC = ROWS // NC
RPT = ROWS_PER_SC // NS
REM = ROWS_PER_SC - NS * RPT
NPAIR = RPT // 2

N_OUT = 10000
N_PAD = 10240
OPT = N_PAD // NS
CROWS = N_PAD // L
ZCHUNK = 32
NCCH = CROWS // 128

RBLK = 1024


def _sc_body(xu, xi, e0, e1, e2, sums, cnts,
             accum, cnt, src_idx, dst_idx, rows0, rows1, hist, iota_idx,
             zrows, zcnt, gsem0, gsem1, ssem0, ssem1):
    c = lax.axis_index("c")
    s = lax.axis_index("s")

    zero16 = jnp.zeros((L,), jnp.float32)
    one16 = jnp.ones((L,), jnp.float32)

    @pl.loop(0, ZCHUNK)
    def _(i):
        @pl.loop(0, D // L)
        def _(j):
            zrows[i, pl.ds(j * L, L)] = zero16

    @pl.loop(0, 128)
    def _(i):
        zcnt[i, :] = zero16

    @pl.loop(0, NCCH)
    def _(j):
        @pl.loop(0, 128 // L)
        def _(m):
            iota_idx[j, pl.ds(m * L, L)] = (
                lax.broadcasted_iota(jnp.int32, (L,), 0) + j * 128 + m * L)

    has_extra = s >= NS - REM
    nb = jnp.where(has_extra, RPT + 1, RPT)
    base_row = c * ROWS_PER_SC + s * RPT
    extra_row = c * ROWS_PER_SC + NS * RPT + (s - (NS - REM))
    my_out = s * OPT

    @pl.loop(0, OPT // ZCHUNK)
    def _(z):
        pltpu.sync_copy(zrows, accum.at[pl.ds(my_out + z * ZCHUNK, ZCHUNK)])

    @pl.loop(0, CROWS)
    def _(z):
        hist[z, :] = zero16

    @pl.when(s < NCCH)
    def _():
        pltpu.sync_copy(zcnt, cnt.at[pl.ds(s * 128, 128)])

    for e, (edges, table) in enumerate(((e0, xu), (e1, xi), (e2, xu))):
        src_hbm = edges.at[0]
        dst_hbm = edges.at[1]

        pltpu.sync_copy(src_hbm.at[pl.ds(base_row, RPT)], src_idx.at[pl.ds(0, RPT)])
        pltpu.sync_copy(dst_hbm.at[pl.ds(base_row, RPT)], dst_idx.at[pl.ds(0, RPT)])

        @pl.when(has_extra)
        def _():
            pltpu.sync_copy(src_hbm.at[pl.ds(extra_row, 1)], src_idx.at[pl.ds(RPT, 1)])
            pltpu.sync_copy(dst_hbm.at[pl.ds(extra_row, 1)], dst_idx.at[pl.ds(RPT, 1)])

        plsc.subcore_barrier()

        def gather(k, buf, sem):
            return pltpu.async_copy(table.at[src_idx.at[k]], buf, sem)

        def gather_wait(k, buf, sem):
            pltpu.make_async_copy(table.at[src_idx.at[k]], buf, sem).wait()

        def drain(k, buf, sem):
            pltpu.async_copy(buf, accum.at[dst_idx.at[k]], sem, add=True)
            for g in range(B // L):
                d = dst_idx[k, pl.ds(g * L, L)]
                plsc.addupdate_scatter(
                    hist, [lax.shift_right_logical(d, 4),
                           jnp.bitwise_and(d, 15)], one16)
            pltpu.make_async_copy(buf, accum.at[dst_idx.at[k]], sem).wait()

        gather(0, rows0, gsem0)
        gather(1, rows1, gsem1)

        @pl.loop(0, NPAIR)
        def _(j):
            k0 = 2 * j
            k1 = k0 + 1
            gather_wait(k0, rows0, gsem0)
            drain(k0, rows0, ssem0)

            @pl.when(k0 + 2 < nb)
            def _():
                gather(k0 + 2, rows0, gsem0)

            gather_wait(k1, rows1, gsem1)
            drain(k1, rows1, ssem1)

            @pl.when(k1 + 2 < nb)
            def _():
                gather(k1 + 2, rows1, gsem1)

        @pl.when(has_extra)
        def _():
            gather_wait(RPT, rows0, gsem0)
            drain(RPT, rows0, ssem0)

        @pl.loop(0, NCCH)
        def _(j):
            pltpu.sync_copy(hist.at[pl.ds(j * 128, 128)],
                            cnt.at[iota_idx.at[j]], add=True)

        plsc.subcore_barrier()

        last = e == 2
        nch = OPT // ZCHUNK

        def co_issue(z, sem):
            pltpu.async_copy(accum.at[pl.ds(my_out + z * ZCHUNK, ZCHUNK)],
                             sums.at[e, c, pl.ds(my_out + z * ZCHUNK, ZCHUNK)],
                             sem)

        def co_wait(z, sem):
            pltpu.make_async_copy(
                accum.at[pl.ds(my_out + z * ZCHUNK, ZCHUNK)],
                sums.at[e, c, pl.ds(my_out + z * ZCHUNK, ZCHUNK)],
                sem).wait()

        def co_zero(z):
            if not last:
                pltpu.sync_copy(zrows, accum.at[pl.ds(my_out + z * ZCHUNK, ZCHUNK)])

        co_issue(0, gsem0)
        co_issue(1, gsem1)

        @pl.loop(0, nch // 2)
        def _(p):
            z0 = 2 * p
            z1 = z0 + 1
            co_wait(z0, gsem0)
            co_zero(z0)

            @pl.when(z0 + 2 < nch)
            def _():
                co_issue(z0 + 2, gsem0)

            co_wait(z1, gsem1)
            co_zero(z1)

            @pl.when(z1 + 2 < nch)
            def _():
                co_issue(z1 + 2, gsem1)

        @pl.when(s < NCCH)
        def _():
            pltpu.sync_copy(cnt.at[pl.ds(s * 128, 128)],
                            cnts.at[e, c, pl.ds(s * 128, 128)])
            if not last:
                pltpu.sync_copy(zcnt, cnt.at[pl.ds(s * 128, 128)])

        if not last:
            @pl.loop(0, CROWS)
            def _(z):
                hist[z, :] = zero16


def _combine_body(sums_ref, cnts_ref, wc_ref, bc_ref, wb_ref, bb_ref,
                  wf_ref, bf_ref, out_ref):
    sm = sums_ref[...]
    cl = cnts_ref[...]
    stot = sm[:, 0] + sm[:, 1]
    ct = cl[:, 0] + cl[:, 1]
    g = RBLK // D
    st4 = stot.reshape(3, g, D, D)
    mean = (st4 / jnp.maximum(ct, 1.0)[:, :, :, None]).reshape(3, RBLK, D)
    mask4 = (ct > 0).astype(jnp.float32)[:, :, :, None]

    def lin(e, w_ref, b_ref):
        y = lax.dot_general(mean[e], w_ref[...], (((1,), (1,)), ((), ())),
                            precision=lax.Precision.HIGHEST)
        bias = (b_ref[...][None, :, :] * mask4[e]).reshape(RBLK, D)
        return y + bias

    out_ref[0] = lin(1, wb_ref, bb_ref) + lin(2, wf_ref, bf_ref)
    out_ref[1] = lin(0, wc_ref, bc_ref)


def kernel(x_user, x_item, edge_click, edge_clicked_by, edge_follow,
           W_click, b_click, W_clicked_by, b_clicked_by, W_follow, b_follow):
    e0 = edge_click.astype(jnp.int32).reshape(2, ROWS, B)
    e1 = edge_clicked_by.astype(jnp.int32).reshape(2, ROWS, B)
    e2 = edge_follow.astype(jnp.int32).reshape(2, ROWS, B)

    mesh = plsc.VectorSubcoreMesh(core_axis_name="c", subcore_axis_name="s",
                                  num_cores=NC, num_subcores=NS)
    sums, cnts = pl.kernel(
        _sc_body,
        out_type=(jax.ShapeDtypeStruct((3, NC, N_PAD, D), jnp.float32),
                  jax.ShapeDtypeStruct((3, NC, CROWS, L), jnp.float32)),
        mesh=mesh,
        compiler_params=pltpu.CompilerParams(use_tc_tiling_on_sc=False,
                                             needs_layout_passes=False),
        scratch_types=[
            pltpu.VMEM_SHARED((N_PAD, D), jnp.float32),
            pltpu.VMEM_SHARED((CROWS, L), jnp.float32),
            pltpu.VMEM((RPT + 1, B), jnp.int32),
            pltpu.VMEM((RPT + 1, B), jnp.int32),
            pltpu.VMEM((B, D), jnp.float32),
            pltpu.VMEM((B, D), jnp.float32),
            pltpu.VMEM((CROWS, L), jnp.float32),
            pltpu.VMEM((NCCH, 128), jnp.int32),
            pltpu.VMEM((ZCHUNK, D), jnp.float32),
            pltpu.VMEM((128, L), jnp.float32),
            pltpu.SemaphoreType.DMA,
            pltpu.SemaphoreType.DMA,
            pltpu.SemaphoreType.DMA,
            pltpu.SemaphoreType.DMA,
        ],
    )(x_user, x_item, e0, e1, e2)

    out2 = pl.pallas_call(
        _combine_body,
        grid=(N_PAD // RBLK,),
        in_specs=[
            pl.BlockSpec((3, NC, RBLK, D), lambda g: (0, 0, g, 0)),
            pl.BlockSpec((3, NC, RBLK // D, D), lambda g: (0, 0, g, 0)),
            pl.BlockSpec((D, D), lambda g: (0, 0)),
            pl.BlockSpec((1, D), lambda g: (0, 0)),
            pl.BlockSpec((D, D), lambda g: (0, 0)),
            pl.BlockSpec((1, D), lambda g: (0, 0)),
            pl.BlockSpec((D, D), lambda g: (0, 0)),
            pl.BlockSpec((1, D), lambda g: (0, 0)),
        ],
        out_specs=pl.BlockSpec((2, RBLK, D), lambda g: (0, g, 0)),
        out_shape=jax.ShapeDtypeStruct((2, N_OUT, D), jnp.float32),
    )(sums, cnts.reshape(3, NC, N_PAD // D, D), W_click, b_click.reshape(1, D),
      W_clicked_by, b_clicked_by.reshape(1, D),
      W_follow, b_follow.reshape(1, D))

    return out2.reshape(2 * N_OUT, D)


# speedup vs baseline: 1.0114x; 1.0114x over previous
import jax
import jax.numpy as jnp
from jax import lax
from jax.experimental import pallas as pl
from jax.experimental.pallas import tpu as pltpu
from jax.experimental.pallas import tpu_sc as plsc

N_USER = 10000
N_ITEM = 10000
D = 128
E = 160000

L = 16
NC = 2
NS = 16
B = 80
ROWS = E // B
ROWS_PER_SC = ROWS // NC
RPT = ROWS_PER_SC // NS
REM = ROWS_PER_SC - NS * RPT
NPAIR = RPT // 2

N_OUT = 10000
N_PAD = 10240
OPT = N_PAD // NS
CROWS = N_PAD // L
ZCHUNK = 32
NCCH = CROWS // 128

RBLK = 1024


def _sc_body(xu, xi, e0, e1, e2, sums, cnts,
             accum, cnt, src_idx, dst_idx, rows0, rows1, hist, iota_idx,
             zrows, zcnt, gsem0, gsem1, ssem0, ssem1):
    c = lax.axis_index("c")
    s = lax.axis_index("s")

    zero16 = jnp.zeros((L,), jnp.float32)
    one16 = jnp.ones((L,), jnp.float32)

    @pl.loop(0, ZCHUNK)
    def _(i):
        @pl.loop(0, D // L)
        def _(j):
            zrows[i, pl.ds(j * L, L)] = zero16

    @pl.loop(0, 128)
    def _(i):
        zcnt[i, :] = zero16

    @pl.loop(0, NCCH)
    def _(j):
        @pl.loop(0, 128 // L)
        def _(m):
            iota_idx[j, pl.ds(m * L, L)] = (
                lax.broadcasted_iota(jnp.int32, (L,), 0) + j * 128 + m * L)

    has_extra = s >= NS - REM
    nb = jnp.where(has_extra, RPT + 1, RPT)
    base_row = c * ROWS_PER_SC + s * RPT
    extra_row = c * ROWS_PER_SC + NS * RPT + (s - (NS - REM))
    my_out = s * OPT

    for e, (edges, table) in enumerate(((e0, xu), (e1, xi), (e2, xu))):
        src_hbm = edges.at[0]
        dst_hbm = edges.at[1]

        pltpu.sync_copy(src_hbm.at[pl.ds(base_row, RPT)], src_idx.at[pl.ds(0, RPT)])
        pltpu.sync_copy(dst_hbm.at[pl.ds(base_row, RPT)], dst_idx.at[pl.ds(0, RPT)])

        @pl.when(has_extra)
        def _():
            pltpu.sync_copy(src_hbm.at[pl.ds(extra_row, 1)], src_idx.at[pl.ds(RPT, 1)])
            pltpu.sync_copy(dst_hbm.at[pl.ds(extra_row, 1)], dst_idx.at[pl.ds(RPT, 1)])

        def gather(k, buf, sem):
            return pltpu.async_copy(table.at[src_idx.at[k]], buf, sem)

        def gather_wait(k, buf, sem):
            pltpu.make_async_copy(table.at[src_idx.at[k]], buf, sem).wait()

        def drain(k, buf, sem):
            pltpu.async_copy(buf, accum.at[dst_idx.at[k]], sem, add=True)
            for g in range(B // L):
                d = dst_idx[k, pl.ds(g * L, L)]
                plsc.addupdate_scatter(
                    hist, [lax.shift_right_logical(d, 4),
                           jnp.bitwise_and(d, 15)], one16)
            pltpu.make_async_copy(buf, accum.at[dst_idx.at[k]], sem).wait()

        gather(0, rows0, gsem0)
        gather(1, rows1, gsem1)

        if e == 0:
            @pl.loop(0, OPT // ZCHUNK)
            def _(z):
                pltpu.sync_copy(zrows, accum.at[pl.ds(my_out + z * ZCHUNK, ZCHUNK)])

            @pl.loop(0, CROWS)
            def _(z):
                hist[z, :] = zero16

            @pl.when(s < NCCH)
            def _():
                pltpu.sync_copy(zcnt, cnt.at[pl.ds(s * 128, 128)])

        plsc.subcore_barrier()

        @pl.loop(0, NPAIR)
        def _(j):
            k0 = 2 * j
            k1 = k0 + 1
            gather_wait(k0, rows0, gsem0)
            drain(k0, rows0, ssem0)

            @pl.when(k0 + 2 < nb)
            def _():
                gather(k0 + 2, rows0, gsem0)

            gather_wait(k1, rows1, gsem1)
            drain(k1, rows1, ssem1)

            @pl.when(k1 + 2 < nb)
            def _():
                gather(k1 + 2, rows1, gsem1)

        @pl.when(has_extra)
        def _():
            gather_wait(RPT, rows0, gsem0)
            drain(RPT, rows0, ssem0)

        @pl.loop(0, NCCH)
        def _(j):
            pltpu.sync_copy(hist.at[pl.ds(j * 128, 128)],
                            cnt.at[iota_idx.at[j]], add=True)

        plsc.subcore_barrier()

        last = e == 2
        nch = OPT // ZCHUNK

        def co_issue(z, sem):
            pltpu.async_copy(accum.at[pl.ds(my_out + z * ZCHUNK, ZCHUNK)],
                             sums.at[e, c, pl.ds(my_out + z * ZCHUNK, ZCHUNK)],
                             sem)

        def co_wait(z, sem):
            pltpu.make_async_copy(
                accum.at[pl.ds(my_out + z * ZCHUNK, ZCHUNK)],
                sums.at[e, c, pl.ds(my_out + z * ZCHUNK, ZCHUNK)],
                sem).wait()

        def co_zero(z):
            if not last:
                pltpu.sync_copy(zrows, accum.at[pl.ds(my_out + z * ZCHUNK, ZCHUNK)])

        co_issue(0, gsem0)
        co_issue(1, gsem1)

        @pl.loop(0, nch // 2)
        def _(p):
            z0 = 2 * p
            z1 = z0 + 1
            co_wait(z0, gsem0)
            co_zero(z0)

            @pl.when(z0 + 2 < nch)
            def _():
                co_issue(z0 + 2, gsem0)

            co_wait(z1, gsem1)
            co_zero(z1)

            @pl.when(z1 + 2 < nch)
            def _():
                co_issue(z1 + 2, gsem1)

        @pl.when(s < NCCH)
        def _():
            pltpu.sync_copy(cnt.at[pl.ds(s * 128, 128)],
                            cnts.at[e, c, pl.ds(s * 128, 128)])
            if not last:
                pltpu.sync_copy(zcnt, cnt.at[pl.ds(s * 128, 128)])

        if not last:
            @pl.loop(0, CROWS)
            def _(z):
                hist[z, :] = zero16


def _combine_body(sums_ref, cnts_ref, wc_ref, bc_ref, wb_ref, bb_ref,
                  wf_ref, bf_ref, out_ref):
    sm = sums_ref[...]
    cl = cnts_ref[...]
    stot = sm[:, 0] + sm[:, 1]
    ct = cl[:, 0] + cl[:, 1]
    g = RBLK // D
    st4 = stot.reshape(3, g, D, D)
    mean = (st4 / jnp.maximum(ct, 1.0)[:, :, :, None]).reshape(3, RBLK, D)
    mask4 = (ct > 0).astype(jnp.float32)[:, :, :, None]

    def lin(e, w_ref, b_ref):
        y = lax.dot_general(mean[e], w_ref[...], (((1,), (1,)), ((), ())),
                            precision=lax.Precision.HIGHEST)
        bias = (b_ref[...][None, :, :] * mask4[e]).reshape(RBLK, D)
        return y + bias

    out_ref[0] = lin(1, wb_ref, bb_ref) + lin(2, wf_ref, bf_ref)
    out_ref[1] = lin(0, wc_ref, bc_ref)


def kernel(x_user, x_item, edge_click, edge_clicked_by, edge_follow,
           W_click, b_click, W_clicked_by, b_clicked_by, W_follow, b_follow):
    e0 = edge_click.astype(jnp.int32).reshape(2, ROWS, B)
    e1 = edge_clicked_by.astype(jnp.int32).reshape(2, ROWS, B)
    e2 = edge_follow.astype(jnp.int32).reshape(2, ROWS, B)

    mesh = plsc.VectorSubcoreMesh(core_axis_name="c", subcore_axis_name="s",
                                  num_cores=NC, num_subcores=NS)
    sums, cnts = pl.kernel(
        _sc_body,
        out_type=(jax.ShapeDtypeStruct((3, NC, N_PAD, D), jnp.float32),
                  jax.ShapeDtypeStruct((3, NC, CROWS, L), jnp.float32)),
        mesh=mesh,
        compiler_params=pltpu.CompilerParams(use_tc_tiling_on_sc=False,
                                             needs_layout_passes=False),
        scratch_types=[
            pltpu.VMEM_SHARED((N_PAD, D), jnp.float32),
            pltpu.VMEM_SHARED((CROWS, L), jnp.float32),
            pltpu.VMEM((RPT + 1, B), jnp.int32),
            pltpu.VMEM((RPT + 1, B), jnp.int32),
            pltpu.VMEM((B, D), jnp.float32),
            pltpu.VMEM((B, D), jnp.float32),
            pltpu.VMEM((CROWS, L), jnp.float32),
            pltpu.VMEM((NCCH, 128), jnp.int32),
            pltpu.VMEM((ZCHUNK, D), jnp.float32),
            pltpu.VMEM((128, L), jnp.float32),
            pltpu.SemaphoreType.DMA,
            pltpu.SemaphoreType.DMA,
            pltpu.SemaphoreType.DMA,
            pltpu.SemaphoreType.DMA,
        ],
    )(x_user, x_item, e0, e1, e2)

    out2 = pl.pallas_call(
        _combine_body,
        grid=(N_PAD // RBLK,),
        in_specs=[
            pl.BlockSpec((3, NC, RBLK, D), lambda g: (0, 0, g, 0)),
            pl.BlockSpec((3, NC, RBLK // D, D), lambda g: (0, 0, g, 0)),
            pl.BlockSpec((D, D), lambda g: (0, 0)),
            pl.BlockSpec((1, D), lambda g: (0, 0)),
            pl.BlockSpec((D, D), lambda g: (0, 0)),
            pl.BlockSpec((1, D), lambda g: (0, 0)),
            pl.BlockSpec((D, D), lambda g: (0, 0)),
            pl.BlockSpec((1, D), lambda g: (0, 0)),
        ],
        out_specs=pl.BlockSpec((2, RBLK, D), lambda g: (0, g, 0)),
        out_shape=jax.ShapeDtypeStruct((2, N_OUT, D), jnp.float32),
    )(sums, cnts.reshape(3, NC, N_PAD // D, D), W_click, b_click.reshape(1, D),
      W_clicked_by, b_clicked_by.reshape(1, D),
      W_follow, b_follow.reshape(1, D))

    return out2.reshape(2 * N_OUT, D)
